# Initial kernel scaffold; baseline (speedup 1.0000x reference)
import jax
import jax.numpy as jnp
from jax.experimental import pallas as pl


def kernel(x, edge_index, edge_attr, params):
    raise NotImplementedError("write your pallas kernel here")


# speedup vs baseline: 1.5992x; 1.5992x over previous
import functools

import jax
import jax.numpy as jnp
from jax import lax
from jax.experimental import pallas as pl
from jax.experimental.pallas import tpu as pltpu
from jax.experimental.pallas import tpu_sc as plsc

N = 10000
E = 320000
D = 128

NC = 2
NS = 16
NW = NC * NS
EPW = E // NW
ET = 40
NCHUNK = EPW // ET
RB = 80
NBLK = N // RB


def _proj_body(x_ref, w_ref, b_ref, k_ref, qv_ref, s_ref):
    acc = jnp.dot(x_ref[...], w_ref[...], preferred_element_type=jnp.float32)
    acc = acc + b_ref[...]
    k_ref[...] = acc[:, :D]
    qv_ref[...] = acc[:, D:3 * D]
    s_ref[...] = acc[:, 3 * D:]


def _proj(x, w, b):
    blk = 1000
    grid = N // blk
    return pl.pallas_call(
        _proj_body,
        grid=(grid,),
        in_specs=[
            pl.BlockSpec((blk, D), lambda i: (i, 0)),
            pl.BlockSpec((D, 4 * D), lambda i: (0, 0)),
            pl.BlockSpec((1, 4 * D), lambda i: (0, 0)),
        ],
        out_specs=[
            pl.BlockSpec((blk, D), lambda i: (i, 0)),
            pl.BlockSpec((blk, 2 * D), lambda i: (i, 0)),
            pl.BlockSpec((blk, D), lambda i: (i, 0)),
        ],
        out_shape=[
            jax.ShapeDtypeStruct((N, D), jnp.float32),
            jax.ShapeDtypeStruct((N, 2 * D), jnp.float32),
            jax.ShapeDtypeStruct((N, D), jnp.float32),
        ],
    )(x, w, b)


def _comb_proj_body(agg_ref, skip_ref, w_ref, b_ref, k_ref, qv_ref, s_ref):
    h = agg_ref[0] + agg_ref[1] + skip_ref[...]
    acc = jnp.dot(h, w_ref[...], preferred_element_type=jnp.float32)
    acc = acc + b_ref[...]
    k_ref[...] = acc[:, :D]
    qv_ref[...] = acc[:, D:3 * D]
    s_ref[...] = acc[:, 3 * D:]


def _comb_proj(agg, skip, w, b):
    blk = 1000
    grid = N // blk
    return pl.pallas_call(
        _comb_proj_body,
        grid=(grid,),
        in_specs=[
            pl.BlockSpec((2, blk, D), lambda i: (0, i, 0)),
            pl.BlockSpec((blk, D), lambda i: (i, 0)),
            pl.BlockSpec((D, 4 * D), lambda i: (0, 0)),
            pl.BlockSpec((1, 4 * D), lambda i: (0, 0)),
        ],
        out_specs=[
            pl.BlockSpec((blk, D), lambda i: (i, 0)),
            pl.BlockSpec((blk, 2 * D), lambda i: (i, 0)),
            pl.BlockSpec((blk, D), lambda i: (i, 0)),
        ],
        out_shape=[
            jax.ShapeDtypeStruct((N, D), jnp.float32),
            jax.ShapeDtypeStruct((N, 2 * D), jnp.float32),
            jax.ShapeDtypeStruct((N, D), jnp.float32),
        ],
    )(agg, skip, w, b)


def _set2set_body(agg_ref, skip_ref, wih_ref, whh_ref, bg_ref, wo_ref,
                  bo_ref, out_ref):
    hx = agg_ref[0] + agg_ref[1] + skip_ref[...]
    h = jnp.zeros((1, D), jnp.float32)
    c = jnp.zeros((1, D), jnp.float32)
    q_star = jnp.zeros((1, 2 * D), jnp.float32)
    for _ in range(3):
        gates = (
            lax.dot_general(q_star, wih_ref[...], (((1,), (1,)), ((), ())),
                            preferred_element_type=jnp.float32)
            + lax.dot_general(h, whh_ref[...], (((1,), (1,)), ((), ())),
                              preferred_element_type=jnp.float32)
            + bg_ref[...]
        )
        i_g = jax.nn.sigmoid(gates[:, :D])
        f_g = jax.nn.sigmoid(gates[:, D:2 * D])
        g_g = jnp.tanh(gates[:, 2 * D:3 * D])
        o_g = jax.nn.sigmoid(gates[:, 3 * D:])
        c = f_g * c + i_g * g_g
        h = o_g * jnp.tanh(c)
        e = lax.dot_general(h, hx, (((1,), (1,)), ((), ())),
                            preferred_element_type=jnp.float32)
        m = jnp.max(e, axis=1, keepdims=True)
        p = jnp.exp(e - m)
        a = p / jnp.sum(p, axis=1, keepdims=True)
        r = jnp.dot(a, hx, preferred_element_type=jnp.float32)
        q_star = jnp.concatenate([h, r], axis=1)
    out_ref[...] = (
        lax.dot_general(q_star, wo_ref[...], (((1,), (1,)), ((), ())),
                        preferred_element_type=jnp.float32)
        + bo_ref[...]
    )


def _set2set(agg, skip, wih, whh, bg, wo, bo):
    return pl.pallas_call(
        _set2set_body,
        out_shape=jax.ShapeDtypeStruct((1, D), jnp.float32),
    )(agg, skip, wih, whh, bg, wo, bo)


def _edge_body(k_hbm, qv_hbm, src_hbm, dst_hbm, out_hbm,
               sidx0, didx0, kd0, qv0, sidx1, didx1, kd1, qv1, zbuf,
               aggr, gk0, gq0, gk1, gq1):
    cid = lax.axis_index("c")
    sid = lax.axis_index("s")
    wid = cid * NS + sid

    sidx = (sidx0, sidx1)
    didx = (didx0, didx1)
    kd = (kd0, kd1)
    qv = (qv0, qv1)
    gk = (gk0, gk1)
    gq = (gq0, gq1)

    def _zr(i, _):
        for j in range(8):
            zbuf[i, pl.ds(j * 16, 16)] = jnp.zeros((16,), jnp.float32)
        return 0
    lax.fori_loop(0, RB, _zr, 0)
    for t in range((NBLK + NS - 1) // NS):
        blk = t * NS + sid

        @pl.when(blk < NBLK)
        def _():
            pltpu.sync_copy(zbuf, aggr.at[pl.ds(blk * RB, RB), :])
    plsc.subcore_barrier()

    def _fire(c, b):
        base = wid * EPW + c * ET
        pltpu.sync_copy(dst_hbm.at[pl.ds(base, ET)], didx[b])
        pltpu.sync_copy(src_hbm.at[pl.ds(base, ET)], sidx[b])
        pltpu.async_copy(k_hbm.at[didx[b]], kd[b], gk[b])
        pltpu.async_copy(qv_hbm.at[sidx[b]], qv[b], gq[b])

    for b in range(2):
        _fire(b, b)

    def _pair(c2, _):
        for b in range(2):
            c = c2 * 2 + b
            pltpu.make_async_copy(k_hbm.at[didx[b]], kd[b], gk[b]).wait()
            pltpu.make_async_copy(qv_hbm.at[sidx[b]], qv[b], gq[b]).wait()

            def _row(i, _i):
                for j in range(8):
                    sl = pl.ds(j * 16, 16)
                    z = kd[b][i, sl] + qv[b][i, sl]
                    eta = 1.0 / (1.0 + jnp.exp(-z))
                    kd[b][i, sl] = eta * qv[b][i, pl.ds(D + j * 16, 16)]
                return 0
            lax.fori_loop(0, ET, _row, 0, unroll=2)
            pltpu.sync_copy(kd[b], aggr.at[didx[b]], add=True)

            @pl.when(c + 2 < NCHUNK)
            def _():
                _fire(c + 2, b)
        return 0
    lax.fori_loop(0, NCHUNK // 2, _pair, 0)
    plsc.subcore_barrier()

    for t in range((NBLK + NS - 1) // NS):
        blk = t * NS + sid

        @pl.when(blk < NBLK)
        def _():
            r0 = blk * RB
            pltpu.sync_copy(aggr.at[pl.ds(r0, RB), :], zbuf)
            pltpu.sync_copy(zbuf, out_hbm.at[cid, pl.ds(r0, RB), :])


@functools.lru_cache(maxsize=1)
def _make_edge_kernel():
    return pl.kernel(
        _edge_body,
        out_type=jax.ShapeDtypeStruct((NC, N, D), jnp.float32),
        mesh=plsc.VectorSubcoreMesh(core_axis_name="c", subcore_axis_name="s"),
        scratch_types=(
            [pltpu.VMEM((ET,), jnp.int32),
             pltpu.VMEM((ET,), jnp.int32),
             pltpu.VMEM((ET, D), jnp.float32),
             pltpu.VMEM((ET, 2 * D), jnp.float32)] * 2
            + [pltpu.VMEM((RB, D), jnp.float32),
               pltpu.VMEM_SHARED((N, D), jnp.float32)]
            + [pltpu.SemaphoreType.DMA] * 4
        ),
    )


def _edge_kernel(k, qv, src, dst):
    return _make_edge_kernel()(k, qv, src, dst)


def _layer_weights(p):
    w = jnp.concatenate(
        [p["key_w"].T, p["query_w"].T, p["value_w"].T, p["skip_w"].T], axis=1)
    b = jnp.concatenate(
        [p["key_b"], p["query_b"], p["value_b"], p["skip_b"]])[None, :]
    return w, b


def kernel(x, edge_index, edge_attr, params):
    src = edge_index[0].astype(jnp.int32)
    dst = edge_index[1].astype(jnp.int32)

    w1, b1 = _layer_weights(params["convs"][0])
    w2, b2 = _layer_weights(params["convs"][1])

    k1, qv1, skip1 = _proj(x, w1, b1)
    agg1 = _edge_kernel(k1, qv1, src, dst)
    k2, qv2, skip2 = _comb_proj(agg1, skip1, w2, b2)
    agg2 = _edge_kernel(k2, qv2, src, dst)

    lstm = params["lstm"]
    bg = (lstm["b_ih"] + lstm["b_hh"])[None, :]
    out = _set2set(agg2, skip2, lstm["w_ih"], lstm["w_hh"], bg,
                   params["out_w"], params["out_b"][None, :])
    return out
